# Initial kernel scaffold; baseline (speedup 1.0000x reference)
import jax
import jax.numpy as jnp
from jax.experimental import pallas as pl


def kernel(noise_params, edge_index, W_emb, b_emb, W1, b1, W2, b2, W3, b3, W_out, b_out):
    raise NotImplementedError("write your pallas kernel here")


# speedup vs baseline: 16.5882x; 16.5882x over previous
import functools
import jax
import jax.numpy as jnp
from jax import lax
from jax.experimental import pallas as pl
from jax.experimental.pallas import tpu as pltpu
from jax.experimental.pallas import tpu_sc as plsc

N = 50000
H = 64
HH = 32
E = 800000
EB = 128
NB = 6272
E_PAD = NB * EB
NACC = 50048
BPT_MSG = NB // 16
BPT_DEG = NB // 32
GRP = 56
ROWS_A = 3128
ROWS_B = N - 15 * ROWS_A

_mesh = plsc.VectorSubcoreMesh(core_axis_name="c", subcore_axis_name="s")
_sc_params = pltpu.CompilerParams(use_tc_tiling_on_sc=False)


@functools.partial(
    pl.kernel,
    out_type=(
        jax.ShapeDtypeStruct((N, 16), jnp.float32),
        jax.ShapeDtypeStruct((N, 16), jnp.float32),
    ),
    mesh=_mesh,
    compiler_params=_sc_params,
    scratch_types=[
        pltpu.VMEM((BPT_DEG, EB), jnp.int32),
        pltpu.VMEM((EB, 16), jnp.float32),
        pltpu.VMEM((ROWS_A // 4, 16), jnp.float32),
        pltpu.VMEM_SHARED((NACC, 16), jnp.float32),
    ],
)
def _deg_kernel(dst4, dega, degb, dstbuf, ones_v, zbuf, acc):
    c = lax.axis_index("c")
    s = lax.axis_index("s")
    wid = c * 16 + s

    def fill(i, _):
        ones_v[i, :] = jnp.full((16,), 1.0, jnp.float32)
        return 0

    lax.fori_loop(0, EB, fill, 0)

    def zfill(i, _):
        zbuf[i, :] = jnp.zeros((16,), jnp.float32)
        return 0

    lax.fori_loop(0, ROWS_A // 4, zfill, 0)
    for k in range(4):
        pltpu.sync_copy(zbuf, acc.at[pl.ds(s * ROWS_A + k * (ROWS_A // 4), ROWS_A // 4)])
    plsc.subcore_barrier()

    pltpu.sync_copy(dst4.at[wid], dstbuf)

    def body(b, _):
        pltpu.sync_copy(ones_v, acc.at[dstbuf.at[b]], add=True)
        return 0

    lax.fori_loop(0, BPT_DEG, body, 0)
    plsc.subcore_barrier()

    def drain(out_ref):
        @pl.when(s < 15)
        def _():
            off = s * ROWS_A
            pltpu.sync_copy(acc.at[pl.ds(off, ROWS_A)], out_ref.at[pl.ds(off, ROWS_A)])

        @pl.when(s == 15)
        def _():
            off = 15 * ROWS_A
            pltpu.sync_copy(acc.at[pl.ds(off, ROWS_B)], out_ref.at[pl.ds(off, ROWS_B)])

    @pl.when(c == 0)
    def _():
        drain(dega)

    @pl.when(c == 1)
    def _():
        drain(degb)


@functools.partial(
    pl.kernel,
    out_type=(
        jax.ShapeDtypeStruct((N, HH), jnp.float32),
        jax.ShapeDtypeStruct((N, HH), jnp.float32),
    ),
    mesh=_mesh,
    compiler_params=_sc_params,
    scratch_types=[
        pltpu.VMEM((GRP, EB), jnp.int32),
        pltpu.VMEM((GRP, EB), jnp.int32),
        pltpu.VMEM((EB, HH), jnp.float32),
        pltpu.VMEM_SHARED((NACC, HH), jnp.float32),
        pltpu.SemaphoreType.DMA,
    ],
)
def _msg_kernel(hpa, hpb, src2, dst3, outa, outb, srcbuf, dstbuf, rows, acc, sem):
    c = lax.axis_index("c")
    s = lax.axis_index("s")

    def run(hp_ref, out_ref):
        @pl.when(s < 15)
        def _():
            off = s * ROWS_A
            pltpu.sync_copy(hp_ref.at[pl.ds(off, ROWS_A)], acc.at[pl.ds(off, ROWS_A)])

        @pl.when(s == 15)
        def _():
            off = 15 * ROWS_A
            pltpu.sync_copy(hp_ref.at[pl.ds(off, ROWS_B)], acc.at[pl.ds(off, ROWS_B)])

        plsc.subcore_barrier()

        base = s * BPT_MSG

        def body(b, _):
            pltpu.async_copy(hp_ref.at[srcbuf.at[b]], rows, sem).wait()
            pltpu.sync_copy(rows, acc.at[dstbuf.at[b]], add=True)
            return 0

        for g in range(BPT_MSG // GRP):
            pltpu.sync_copy(src2.at[pl.ds(base + g * GRP, GRP)], srcbuf)
            pltpu.sync_copy(dst3.at[pl.ds(base + g * GRP, GRP)], dstbuf)
            lax.fori_loop(0, GRP, body, 0)
        plsc.subcore_barrier()

        @pl.when(s < 15)
        def _():
            off = s * ROWS_A
            pltpu.sync_copy(acc.at[pl.ds(off, ROWS_A)], out_ref.at[pl.ds(off, ROWS_A)])

        @pl.when(s == 15)
        def _():
            off = 15 * ROWS_A
            pltpu.sync_copy(acc.at[pl.ds(off, ROWS_B)], out_ref.at[pl.ds(off, ROWS_B)])

    @pl.when(c == 0)
    def _():
        run(hpa, outa)

    @pl.when(c == 1)
    def _():
        run(hpb, outb)


BN = 2000
GRID = N // BN


def _embed_body(noise, dega, degb, w_emb, b_emb, w1, hpa, hpb, dis_o):
    deg = dega[:, 0:1] + degb[:, 0:1] + 1.0
    dis = lax.rsqrt(deg)
    x0 = jnp.maximum(jnp.dot(noise[...], w_emb[...],
                             preferred_element_type=jnp.float32) + b_emb[...], 0.0)
    hp = dis * jnp.dot(x0, w1[...], preferred_element_type=jnp.float32)
    hpa[...] = hp[:, :HH]
    hpb[...] = hp[:, HH:]
    dis_o[...] = dis


def _combine_body(sa, sb, dis, b_l, w_next, hpa, hpb):
    svec = jnp.concatenate([sa[...], sb[...]], axis=1)
    x = jnp.maximum(dis[...] * svec + b_l[...], 0.0)
    hp = dis[...] * jnp.dot(x, w_next[...], preferred_element_type=jnp.float32)
    hpa[...] = hp[:, :HH]
    hpb[...] = hp[:, HH:]


def _final_body(sa, sb, dis, b_l, w_out, b_out, ang):
    svec = jnp.concatenate([sa[...], sb[...]], axis=1)
    x = jnp.maximum(dis[...] * svec + b_l[...], 0.0)
    y = jnp.dot(x, w_out[...], preferred_element_type=jnp.float32) + b_out[...]
    ang[...] = jnp.tanh(y) * jnp.pi


def _row_spec(width):
    return pl.BlockSpec((BN, width), lambda i: (i, 0))


def _full_spec(shape):
    return pl.BlockSpec(shape, lambda i: tuple(0 for _ in shape))


def kernel(noise_params, edge_index, W_emb, b_emb, W1, b1, W2, b2, W3, b3, W_out, b_out):
    src = edge_index[0]
    dst = edge_index[1]
    pad = E_PAD - E
    src2 = jnp.concatenate([src, jnp.zeros((pad,), jnp.int32)]).reshape(NB, EB)
    dst3 = jnp.concatenate([dst, jnp.full((pad,), N, jnp.int32)]).reshape(NB, EB)

    dega, degb = _deg_kernel(dst3.reshape(32, BPT_DEG, EB))

    noise = noise_params.reshape(N, 4)
    embed = pl.pallas_call(
        _embed_body,
        grid=(GRID,),
        in_specs=[
            _row_spec(4), _row_spec(16), _row_spec(16),
            _full_spec((4, H)), _full_spec((1, H)), _full_spec((H, H)),
        ],
        out_specs=[_row_spec(HH), _row_spec(HH), _row_spec(1)],
        out_shape=[
            jax.ShapeDtypeStruct((N, HH), jnp.float32),
            jax.ShapeDtypeStruct((N, HH), jnp.float32),
            jax.ShapeDtypeStruct((N, 1), jnp.float32),
        ],
    )
    hpa, hpb, dis = embed(noise, dega, degb, W_emb, b_emb.reshape(1, H), W1)

    combine = pl.pallas_call(
        _combine_body,
        grid=(GRID,),
        in_specs=[
            _row_spec(HH), _row_spec(HH), _row_spec(1),
            _full_spec((1, H)), _full_spec((H, H)),
        ],
        out_specs=[_row_spec(HH), _row_spec(HH)],
        out_shape=[
            jax.ShapeDtypeStruct((N, HH), jnp.float32),
            jax.ShapeDtypeStruct((N, HH), jnp.float32),
        ],
    )

    sa, sb = _msg_kernel(hpa, hpb, src2, dst3)
    hpa, hpb = combine(sa, sb, dis, b1.reshape(1, H), W2)
    sa, sb = _msg_kernel(hpa, hpb, src2, dst3)
    hpa, hpb = combine(sa, sb, dis, b2.reshape(1, H), W3)
    sa, sb = _msg_kernel(hpa, hpb, src2, dst3)

    final = pl.pallas_call(
        _final_body,
        grid=(GRID,),
        in_specs=[
            _row_spec(HH), _row_spec(HH), _row_spec(1),
            _full_spec((1, H)), _full_spec((H, 3)), _full_spec((1, 3)),
        ],
        out_specs=[_row_spec(3)],
        out_shape=[jax.ShapeDtypeStruct((N, 3), jnp.float32)],
    )
    (ang,) = final(sa, sb, dis, b3.reshape(1, H), W_out, b_out.reshape(1, 3))
    return ang.reshape(1, N, 3)


# speedup vs baseline: 19.0101x; 1.1460x over previous
import functools
import jax
import jax.numpy as jnp
from jax import lax
from jax.experimental import pallas as pl
from jax.experimental.pallas import tpu as pltpu
from jax.experimental.pallas import tpu_sc as plsc

N = 50000
H = 64
HH = 32
E = 800000
EB = 128
NB = 6272
E_PAD = NB * EB
NACC = 50048
BPT_MSG = NB // 16
BPT_DEG = NB // 32
GRP = 56
ROWS_A = 3128
ROWS_B = N - 15 * ROWS_A

_mesh = plsc.VectorSubcoreMesh(core_axis_name="c", subcore_axis_name="s")
_sc_params = pltpu.CompilerParams(use_tc_tiling_on_sc=False)


@functools.partial(
    pl.kernel,
    out_type=(
        jax.ShapeDtypeStruct((N, 16), jnp.float32),
        jax.ShapeDtypeStruct((N, 16), jnp.float32),
    ),
    mesh=_mesh,
    compiler_params=_sc_params,
    scratch_types=[
        pltpu.VMEM((BPT_DEG, EB), jnp.int32),
        pltpu.VMEM((EB, 16), jnp.float32),
        pltpu.VMEM((ROWS_A // 4, 16), jnp.float32),
        pltpu.VMEM_SHARED((NACC, 16), jnp.float32),
    ],
)
def _deg_kernel(dst4, dega, degb, dstbuf, ones_v, zbuf, acc):
    c = lax.axis_index("c")
    s = lax.axis_index("s")
    wid = c * 16 + s

    def fill(i, _):
        ones_v[i, :] = jnp.full((16,), 1.0, jnp.float32)
        return 0

    lax.fori_loop(0, EB, fill, 0)

    def zfill(i, _):
        zbuf[i, :] = jnp.zeros((16,), jnp.float32)
        return 0

    lax.fori_loop(0, ROWS_A // 4, zfill, 0)
    for k in range(4):
        pltpu.sync_copy(zbuf, acc.at[pl.ds(s * ROWS_A + k * (ROWS_A // 4), ROWS_A // 4)])
    plsc.subcore_barrier()

    pltpu.sync_copy(dst4.at[wid], dstbuf)

    def body(b, _):
        pltpu.sync_copy(ones_v, acc.at[dstbuf.at[b]], add=True)
        return 0

    lax.fori_loop(0, BPT_DEG, body, 0)
    plsc.subcore_barrier()

    def drain(out_ref):
        @pl.when(s < 15)
        def _():
            off = s * ROWS_A
            pltpu.sync_copy(acc.at[pl.ds(off, ROWS_A)], out_ref.at[pl.ds(off, ROWS_A)])

        @pl.when(s == 15)
        def _():
            off = 15 * ROWS_A
            pltpu.sync_copy(acc.at[pl.ds(off, ROWS_B)], out_ref.at[pl.ds(off, ROWS_B)])

    @pl.when(c == 0)
    def _():
        drain(dega)

    @pl.when(c == 1)
    def _():
        drain(degb)


@functools.partial(
    pl.kernel,
    out_type=(
        jax.ShapeDtypeStruct((N, HH), jnp.float32),
        jax.ShapeDtypeStruct((N, HH), jnp.float32),
    ),
    mesh=_mesh,
    compiler_params=_sc_params,
    scratch_types=[
        pltpu.VMEM((GRP, EB), jnp.int32),
        pltpu.VMEM((GRP, EB), jnp.int32),
        pltpu.VMEM((EB, HH), jnp.float32),
        pltpu.VMEM((EB, HH), jnp.float32),
        pltpu.VMEM_SHARED((NACC, HH), jnp.float32),
        pltpu.SemaphoreType.DMA,
    ],
)
def _msg_kernel(hpa, hpb, src2, dst3, outa, outb, srcbuf, dstbuf, rows0, rows1, acc, sem):
    c = lax.axis_index("c")
    s = lax.axis_index("s")

    def run(hp_ref, out_ref):
        @pl.when(s < 15)
        def _():
            off = s * ROWS_A
            pltpu.sync_copy(hp_ref.at[pl.ds(off, ROWS_A)], acc.at[pl.ds(off, ROWS_A)])

        @pl.when(s == 15)
        def _():
            off = 15 * ROWS_A
            pltpu.sync_copy(hp_ref.at[pl.ds(off, ROWS_B)], acc.at[pl.ds(off, ROWS_B)])

        plsc.subcore_barrier()

        base = s * BPT_MSG

        def pair(p, _):
            b0 = 2 * p
            pltpu.make_async_copy(hp_ref.at[srcbuf.at[b0]], rows0, sem).wait()
            pltpu.async_copy(hp_ref.at[srcbuf.at[b0 + 1]], rows1, sem)
            pltpu.sync_copy(rows0, acc.at[dstbuf.at[b0]], add=True)
            pltpu.make_async_copy(hp_ref.at[srcbuf.at[b0 + 1]], rows1, sem).wait()

            @pl.when(p < GRP // 2 - 1)
            def _():
                pltpu.async_copy(hp_ref.at[srcbuf.at[b0 + 2]], rows0, sem)

            pltpu.sync_copy(rows1, acc.at[dstbuf.at[b0 + 1]], add=True)
            return 0

        for g in range(BPT_MSG // GRP):
            pltpu.sync_copy(src2.at[pl.ds(base + g * GRP, GRP)], srcbuf)
            pltpu.sync_copy(dst3.at[pl.ds(base + g * GRP, GRP)], dstbuf)
            pltpu.async_copy(hp_ref.at[srcbuf.at[0]], rows0, sem)
            lax.fori_loop(0, GRP // 2, pair, 0)
        plsc.subcore_barrier()

        @pl.when(s < 15)
        def _():
            off = s * ROWS_A
            pltpu.sync_copy(acc.at[pl.ds(off, ROWS_A)], out_ref.at[pl.ds(off, ROWS_A)])

        @pl.when(s == 15)
        def _():
            off = 15 * ROWS_A
            pltpu.sync_copy(acc.at[pl.ds(off, ROWS_B)], out_ref.at[pl.ds(off, ROWS_B)])

    @pl.when(c == 0)
    def _():
        run(hpa, outa)

    @pl.when(c == 1)
    def _():
        run(hpb, outb)


BN = 2000
GRID = N // BN


def _embed_body(noise, dega, degb, w_emb, b_emb, w1, hpa, hpb, dis_o):
    deg = dega[:, 0:1] + degb[:, 0:1] + 1.0
    dis = lax.rsqrt(deg)
    x0 = jnp.maximum(jnp.dot(noise[...], w_emb[...],
                             preferred_element_type=jnp.float32) + b_emb[...], 0.0)
    hp = dis * jnp.dot(x0, w1[...], preferred_element_type=jnp.float32)
    hpa[...] = hp[:, :HH]
    hpb[...] = hp[:, HH:]
    dis_o[...] = dis


def _combine_body(sa, sb, dis, b_l, w_next, hpa, hpb):
    svec = jnp.concatenate([sa[...], sb[...]], axis=1)
    x = jnp.maximum(dis[...] * svec + b_l[...], 0.0)
    hp = dis[...] * jnp.dot(x, w_next[...], preferred_element_type=jnp.float32)
    hpa[...] = hp[:, :HH]
    hpb[...] = hp[:, HH:]


def _final_body(sa, sb, dis, b_l, w_out, b_out, ang):
    svec = jnp.concatenate([sa[...], sb[...]], axis=1)
    x = jnp.maximum(dis[...] * svec + b_l[...], 0.0)
    y = jnp.dot(x, w_out[...], preferred_element_type=jnp.float32) + b_out[...]
    ang[...] = jnp.tanh(y) * jnp.pi


def _row_spec(width):
    return pl.BlockSpec((BN, width), lambda i: (i, 0))


def _full_spec(shape):
    return pl.BlockSpec(shape, lambda i: tuple(0 for _ in shape))


def kernel(noise_params, edge_index, W_emb, b_emb, W1, b1, W2, b2, W3, b3, W_out, b_out):
    src = edge_index[0]
    dst = edge_index[1]
    pad = E_PAD - E
    src2 = jnp.concatenate([src, jnp.zeros((pad,), jnp.int32)]).reshape(NB, EB)
    dst3 = jnp.concatenate([dst, jnp.full((pad,), N, jnp.int32)]).reshape(NB, EB)

    dega, degb = _deg_kernel(dst3.reshape(32, BPT_DEG, EB))

    noise = noise_params.reshape(N, 4)
    embed = pl.pallas_call(
        _embed_body,
        grid=(GRID,),
        in_specs=[
            _row_spec(4), _row_spec(16), _row_spec(16),
            _full_spec((4, H)), _full_spec((1, H)), _full_spec((H, H)),
        ],
        out_specs=[_row_spec(HH), _row_spec(HH), _row_spec(1)],
        out_shape=[
            jax.ShapeDtypeStruct((N, HH), jnp.float32),
            jax.ShapeDtypeStruct((N, HH), jnp.float32),
            jax.ShapeDtypeStruct((N, 1), jnp.float32),
        ],
    )
    hpa, hpb, dis = embed(noise, dega, degb, W_emb, b_emb.reshape(1, H), W1)

    combine = pl.pallas_call(
        _combine_body,
        grid=(GRID,),
        in_specs=[
            _row_spec(HH), _row_spec(HH), _row_spec(1),
            _full_spec((1, H)), _full_spec((H, H)),
        ],
        out_specs=[_row_spec(HH), _row_spec(HH)],
        out_shape=[
            jax.ShapeDtypeStruct((N, HH), jnp.float32),
            jax.ShapeDtypeStruct((N, HH), jnp.float32),
        ],
    )

    sa, sb = _msg_kernel(hpa, hpb, src2, dst3)
    hpa, hpb = combine(sa, sb, dis, b1.reshape(1, H), W2)
    sa, sb = _msg_kernel(hpa, hpb, src2, dst3)
    hpa, hpb = combine(sa, sb, dis, b2.reshape(1, H), W3)
    sa, sb = _msg_kernel(hpa, hpb, src2, dst3)

    final = pl.pallas_call(
        _final_body,
        grid=(GRID,),
        in_specs=[
            _row_spec(HH), _row_spec(HH), _row_spec(1),
            _full_spec((1, H)), _full_spec((H, 3)), _full_spec((1, 3)),
        ],
        out_specs=[_row_spec(3)],
        out_shape=[jax.ShapeDtypeStruct((N, 3), jnp.float32)],
    )
    (ang,) = final(sa, sb, dis, b3.reshape(1, H), W_out, b_out.reshape(1, 3))
    return ang.reshape(1, N, 3)


# speedup vs baseline: 25.4921x; 1.3410x over previous
import functools
import jax
import jax.numpy as jnp
from jax import lax
from jax.experimental import pallas as pl
from jax.experimental.pallas import tpu as pltpu
from jax.experimental.pallas import tpu_sc as plsc

N = 50000
H = 64
HH = 32
E = 800000
EB = 128
NB = 6272
E_PAD = NB * EB
NACC = 50048
BPT_MSG = NB // 16
BPT_DEG = NB // 32
GRP = 56
ROWS_A = 3128
ROWS_B = N - 15 * ROWS_A

_mesh = plsc.VectorSubcoreMesh(core_axis_name="c", subcore_axis_name="s")
_sc_params = pltpu.CompilerParams(use_tc_tiling_on_sc=False)


@functools.partial(
    pl.kernel,
    out_type=(
        jax.ShapeDtypeStruct((N, 16), jnp.float32),
        jax.ShapeDtypeStruct((N, 16), jnp.float32),
    ),
    mesh=_mesh,
    compiler_params=_sc_params,
    scratch_types=[
        pltpu.VMEM((BPT_DEG, EB), jnp.int32),
        pltpu.VMEM((EB, 16), jnp.float32),
        pltpu.VMEM((ROWS_A // 4, 16), jnp.float32),
        pltpu.VMEM_SHARED((NACC, 16), jnp.float32),
    ],
)
def _deg_kernel(dst4, dega, degb, dstbuf, ones_v, zbuf, acc):
    c = lax.axis_index("c")
    s = lax.axis_index("s")
    wid = c * 16 + s

    def fill(i, _):
        ones_v[i, :] = jnp.full((16,), 1.0, jnp.float32)
        return 0

    lax.fori_loop(0, EB, fill, 0)

    def zfill(i, _):
        zbuf[i, :] = jnp.zeros((16,), jnp.float32)
        return 0

    lax.fori_loop(0, ROWS_A // 4, zfill, 0)
    for k in range(4):
        pltpu.sync_copy(zbuf, acc.at[pl.ds(s * ROWS_A + k * (ROWS_A // 4), ROWS_A // 4)])
    plsc.subcore_barrier()

    pltpu.sync_copy(dst4.at[wid], dstbuf)

    def body(b, _):
        pltpu.sync_copy(ones_v, acc.at[dstbuf.at[b]], add=True)
        return 0

    lax.fori_loop(0, BPT_DEG, body, 0)
    plsc.subcore_barrier()

    def drain(out_ref):
        @pl.when(s < 15)
        def _():
            off = s * ROWS_A
            pltpu.sync_copy(acc.at[pl.ds(off, ROWS_A)], out_ref.at[pl.ds(off, ROWS_A)])

        @pl.when(s == 15)
        def _():
            off = 15 * ROWS_A
            pltpu.sync_copy(acc.at[pl.ds(off, ROWS_B)], out_ref.at[pl.ds(off, ROWS_B)])

    @pl.when(c == 0)
    def _():
        drain(dega)

    @pl.when(c == 1)
    def _():
        drain(degb)


@functools.partial(
    pl.kernel,
    out_type=(
        jax.ShapeDtypeStruct((N, HH), jnp.float32),
        jax.ShapeDtypeStruct((N, HH), jnp.float32),
    ),
    mesh=_mesh,
    compiler_params=_sc_params,
    scratch_types=[
        pltpu.VMEM((GRP, EB), jnp.int32),
        pltpu.VMEM((GRP, EB), jnp.int32),
        [pltpu.VMEM((EB, HH), jnp.float32) for _ in range(4)],
        pltpu.VMEM_SHARED((NACC, HH), jnp.float32),
        pltpu.SemaphoreType.DMA,
        pltpu.SemaphoreType.DMA,
    ],
)
def _msg_kernel(hpa, hpb, src2, dst3, outa, outb, srcbuf, dstbuf, rows, acc, gsem, ssem):
    c = lax.axis_index("c")
    s = lax.axis_index("s")

    def run(hp_ref, out_ref):
        @pl.when(s < 15)
        def _():
            off = s * ROWS_A
            pltpu.sync_copy(hp_ref.at[pl.ds(off, ROWS_A)], acc.at[pl.ds(off, ROWS_A)])

        @pl.when(s == 15)
        def _():
            off = 15 * ROWS_A
            pltpu.sync_copy(hp_ref.at[pl.ds(off, ROWS_B)], acc.at[pl.ds(off, ROWS_B)])

        plsc.subcore_barrier()

        base = s * BPT_MSG

        def quad(q, _):
            for j in range(4):
                b = 4 * q + j
                pltpu.make_async_copy(hp_ref.at[srcbuf.at[b]], rows[j], gsem).wait()
                pltpu.async_copy(rows[j], acc.at[dstbuf.at[b]], ssem, add=True)
                if j < 2:
                    @pl.when(q > 0)
                    def _(b=b, j=j):
                        pltpu.make_async_copy(
                            rows[(j + 2) % 4], acc.at[dstbuf.at[b - 2]], ssem).wait()

                    pltpu.async_copy(hp_ref.at[srcbuf.at[b + 2]], rows[(j + 2) % 4], gsem)
                else:
                    pltpu.make_async_copy(
                        rows[(j + 2) % 4], acc.at[dstbuf.at[b - 2]], ssem).wait()

                    @pl.when(q < GRP // 4 - 1)
                    def _(b=b, j=j):
                        pltpu.async_copy(hp_ref.at[srcbuf.at[b + 2]], rows[(j + 2) % 4], gsem)
            return 0

        for g in range(BPT_MSG // GRP):
            pltpu.sync_copy(src2.at[pl.ds(base + g * GRP, GRP)], srcbuf)
            pltpu.sync_copy(dst3.at[pl.ds(base + g * GRP, GRP)], dstbuf)
            pltpu.async_copy(hp_ref.at[srcbuf.at[0]], rows[0], gsem)
            pltpu.async_copy(hp_ref.at[srcbuf.at[1]], rows[1], gsem)
            lax.fori_loop(0, GRP // 4, quad, 0)
            pltpu.make_async_copy(rows[2], acc.at[dstbuf.at[GRP - 2]], ssem).wait()
            pltpu.make_async_copy(rows[3], acc.at[dstbuf.at[GRP - 1]], ssem).wait()
        plsc.subcore_barrier()

        @pl.when(s < 15)
        def _():
            off = s * ROWS_A
            pltpu.sync_copy(acc.at[pl.ds(off, ROWS_A)], out_ref.at[pl.ds(off, ROWS_A)])

        @pl.when(s == 15)
        def _():
            off = 15 * ROWS_A
            pltpu.sync_copy(acc.at[pl.ds(off, ROWS_B)], out_ref.at[pl.ds(off, ROWS_B)])

    @pl.when(c == 0)
    def _():
        run(hpa, outa)

    @pl.when(c == 1)
    def _():
        run(hpb, outb)


BN = 2000
GRID = N // BN


def _embed_body(noise, dega, degb, w_emb, b_emb, w1, hpa, hpb, dis_o):
    deg = dega[:, 0:1] + degb[:, 0:1] + 1.0
    dis = lax.rsqrt(deg)
    x0 = jnp.maximum(jnp.dot(noise[...], w_emb[...],
                             preferred_element_type=jnp.float32) + b_emb[...], 0.0)
    hp = dis * jnp.dot(x0, w1[...], preferred_element_type=jnp.float32)
    hpa[...] = hp[:, :HH]
    hpb[...] = hp[:, HH:]
    dis_o[...] = dis


def _combine_body(sa, sb, dis, b_l, w_next, hpa, hpb):
    svec = jnp.concatenate([sa[...], sb[...]], axis=1)
    x = jnp.maximum(dis[...] * svec + b_l[...], 0.0)
    hp = dis[...] * jnp.dot(x, w_next[...], preferred_element_type=jnp.float32)
    hpa[...] = hp[:, :HH]
    hpb[...] = hp[:, HH:]


def _final_body(sa, sb, dis, b_l, w_out, b_out, ang):
    svec = jnp.concatenate([sa[...], sb[...]], axis=1)
    x = jnp.maximum(dis[...] * svec + b_l[...], 0.0)
    y = jnp.dot(x, w_out[...], preferred_element_type=jnp.float32) + b_out[...]
    ang[...] = jnp.tanh(y) * jnp.pi


def _row_spec(width):
    return pl.BlockSpec((BN, width), lambda i: (i, 0))


def _full_spec(shape):
    return pl.BlockSpec(shape, lambda i: tuple(0 for _ in shape))


def kernel(noise_params, edge_index, W_emb, b_emb, W1, b1, W2, b2, W3, b3, W_out, b_out):
    src = edge_index[0]
    dst = edge_index[1]
    pad = E_PAD - E
    src2 = jnp.concatenate([src, jnp.zeros((pad,), jnp.int32)]).reshape(NB, EB)
    dst3 = jnp.concatenate([dst, jnp.full((pad,), N, jnp.int32)]).reshape(NB, EB)

    dega, degb = _deg_kernel(dst3.reshape(32, BPT_DEG, EB))

    noise = noise_params.reshape(N, 4)
    embed = pl.pallas_call(
        _embed_body,
        grid=(GRID,),
        in_specs=[
            _row_spec(4), _row_spec(16), _row_spec(16),
            _full_spec((4, H)), _full_spec((1, H)), _full_spec((H, H)),
        ],
        out_specs=[_row_spec(HH), _row_spec(HH), _row_spec(1)],
        out_shape=[
            jax.ShapeDtypeStruct((N, HH), jnp.float32),
            jax.ShapeDtypeStruct((N, HH), jnp.float32),
            jax.ShapeDtypeStruct((N, 1), jnp.float32),
        ],
    )
    hpa, hpb, dis = embed(noise, dega, degb, W_emb, b_emb.reshape(1, H), W1)

    combine = pl.pallas_call(
        _combine_body,
        grid=(GRID,),
        in_specs=[
            _row_spec(HH), _row_spec(HH), _row_spec(1),
            _full_spec((1, H)), _full_spec((H, H)),
        ],
        out_specs=[_row_spec(HH), _row_spec(HH)],
        out_shape=[
            jax.ShapeDtypeStruct((N, HH), jnp.float32),
            jax.ShapeDtypeStruct((N, HH), jnp.float32),
        ],
    )

    sa, sb = _msg_kernel(hpa, hpb, src2, dst3)
    hpa, hpb = combine(sa, sb, dis, b1.reshape(1, H), W2)
    sa, sb = _msg_kernel(hpa, hpb, src2, dst3)
    hpa, hpb = combine(sa, sb, dis, b2.reshape(1, H), W3)
    sa, sb = _msg_kernel(hpa, hpb, src2, dst3)

    final = pl.pallas_call(
        _final_body,
        grid=(GRID,),
        in_specs=[
            _row_spec(HH), _row_spec(HH), _row_spec(1),
            _full_spec((1, H)), _full_spec((H, 3)), _full_spec((1, 3)),
        ],
        out_specs=[_row_spec(3)],
        out_shape=[jax.ShapeDtypeStruct((N, 3), jnp.float32)],
    )
    (ang,) = final(sa, sb, dis, b3.reshape(1, H), W_out, b_out.reshape(1, 3))
    return ang.reshape(1, N, 3)


# speedup vs baseline: 25.8610x; 1.0145x over previous
import functools
import jax
import jax.numpy as jnp
from jax import lax
from jax.experimental import pallas as pl
from jax.experimental.pallas import tpu as pltpu
from jax.experimental.pallas import tpu_sc as plsc

N = 50000
H = 64
HH = 32
E = 800000
EB = 128
NB = 6272
E_PAD = NB * EB
NACC = 50008
BPT_MSG = NB // 16
BPT_DEG = NB // 32
GRP = 56
ROWS_A = 3128
ROWS_B = N - 15 * ROWS_A

_mesh = plsc.VectorSubcoreMesh(core_axis_name="c", subcore_axis_name="s")
_sc_params = pltpu.CompilerParams(use_tc_tiling_on_sc=False)


@functools.partial(
    pl.kernel,
    out_type=(
        jax.ShapeDtypeStruct((N, 16), jnp.float32),
        jax.ShapeDtypeStruct((N, 16), jnp.float32),
    ),
    mesh=_mesh,
    compiler_params=_sc_params,
    scratch_types=[
        pltpu.VMEM((BPT_DEG, EB), jnp.int32),
        pltpu.VMEM((EB, 16), jnp.float32),
        pltpu.VMEM((ROWS_A // 4, 16), jnp.float32),
        pltpu.VMEM_SHARED((NACC, 16), jnp.float32),
    ],
)
def _deg_kernel(dst4, dega, degb, dstbuf, ones_v, zbuf, acc):
    c = lax.axis_index("c")
    s = lax.axis_index("s")
    wid = c * 16 + s

    def fill(i, _):
        ones_v[i, :] = jnp.full((16,), 1.0, jnp.float32)
        return 0

    lax.fori_loop(0, EB, fill, 0)

    def zfill(i, _):
        zbuf[i, :] = jnp.zeros((16,), jnp.float32)
        return 0

    lax.fori_loop(0, ROWS_A // 4, zfill, 0)
    for k in range(4):
        pltpu.sync_copy(zbuf, acc.at[pl.ds(s * ROWS_A + k * (ROWS_A // 4), ROWS_A // 4)])
    plsc.subcore_barrier()

    pltpu.sync_copy(dst4.at[wid], dstbuf)

    def body(b, _):
        pltpu.sync_copy(ones_v, acc.at[dstbuf.at[b]], add=True)
        return 0

    lax.fori_loop(0, BPT_DEG, body, 0)
    plsc.subcore_barrier()

    def drain(out_ref):
        @pl.when(s < 15)
        def _():
            off = s * ROWS_A
            pltpu.sync_copy(acc.at[pl.ds(off, ROWS_A)], out_ref.at[pl.ds(off, ROWS_A)])

        @pl.when(s == 15)
        def _():
            off = 15 * ROWS_A
            pltpu.sync_copy(acc.at[pl.ds(off, ROWS_B)], out_ref.at[pl.ds(off, ROWS_B)])

    @pl.when(c == 0)
    def _():
        drain(dega)

    @pl.when(c == 1)
    def _():
        drain(degb)


@functools.partial(
    pl.kernel,
    out_type=(
        jax.ShapeDtypeStruct((N, HH), jnp.float32),
        jax.ShapeDtypeStruct((N, HH), jnp.float32),
    ),
    mesh=_mesh,
    compiler_params=_sc_params,
    scratch_types=[
        pltpu.VMEM((GRP, EB), jnp.int32),
        pltpu.VMEM((GRP, EB), jnp.int32),
        [pltpu.VMEM((EB, HH), jnp.float32) for _ in range(4)],
        pltpu.VMEM_SHARED((NACC, HH), jnp.float32),
        pltpu.SemaphoreType.DMA,
        pltpu.SemaphoreType.DMA,
    ],
)
def _msg_kernel(hpa, hpb, src2, dst3, outa, outb, srcbuf, dstbuf, rows, acc, gsem, ssem):
    c = lax.axis_index("c")
    s = lax.axis_index("s")

    def run(hp_ref, out_ref):
        @pl.when(s < 15)
        def _():
            off = s * ROWS_A
            pltpu.sync_copy(hp_ref.at[pl.ds(off, ROWS_A)], acc.at[pl.ds(off, ROWS_A)])

        @pl.when(s == 15)
        def _():
            off = 15 * ROWS_A
            pltpu.sync_copy(hp_ref.at[pl.ds(off, ROWS_B)], acc.at[pl.ds(off, ROWS_B)])

        plsc.subcore_barrier()

        base = s * BPT_MSG

        def quad(q, _):
            for j in range(4):
                b = 4 * q + j
                pltpu.make_async_copy(hp_ref.at[srcbuf.at[b]], rows[j], gsem).wait()
                pltpu.async_copy(rows[j], acc.at[dstbuf.at[b]], ssem, add=True)
                if j < 2:
                    @pl.when(q > 0)
                    def _(b=b, j=j):
                        pltpu.make_async_copy(
                            rows[(j + 2) % 4], acc.at[dstbuf.at[b - 2]], ssem).wait()

                    pltpu.async_copy(hp_ref.at[srcbuf.at[b + 2]], rows[(j + 2) % 4], gsem)
                else:
                    pltpu.make_async_copy(
                        rows[(j + 2) % 4], acc.at[dstbuf.at[b - 2]], ssem).wait()

                    @pl.when(q < GRP // 4 - 1)
                    def _(b=b, j=j):
                        pltpu.async_copy(hp_ref.at[srcbuf.at[b + 2]], rows[(j + 2) % 4], gsem)
            return 0

        for g in range(BPT_MSG // GRP):
            pltpu.sync_copy(src2.at[pl.ds(base + g * GRP, GRP)], srcbuf)
            pltpu.sync_copy(dst3.at[pl.ds(base + g * GRP, GRP)], dstbuf)
            pltpu.async_copy(hp_ref.at[srcbuf.at[0]], rows[0], gsem)
            pltpu.async_copy(hp_ref.at[srcbuf.at[1]], rows[1], gsem)
            lax.fori_loop(0, GRP // 4, quad, 0)
            pltpu.make_async_copy(rows[2], acc.at[dstbuf.at[GRP - 2]], ssem).wait()
            pltpu.make_async_copy(rows[3], acc.at[dstbuf.at[GRP - 1]], ssem).wait()
        plsc.subcore_barrier()

        @pl.when(s < 15)
        def _():
            off = s * ROWS_A
            pltpu.sync_copy(acc.at[pl.ds(off, ROWS_A)], out_ref.at[pl.ds(off, ROWS_A)])

        @pl.when(s == 15)
        def _():
            off = 15 * ROWS_A
            pltpu.sync_copy(acc.at[pl.ds(off, ROWS_B)], out_ref.at[pl.ds(off, ROWS_B)])

    @pl.when(c == 0)
    def _():
        run(hpa, outa)

    @pl.when(c == 1)
    def _():
        run(hpb, outb)


BN = 5000
GRID = N // BN


def _embed_body(noise, dega, degb, w_emb, b_emb, w1, hpa, hpb, dis_o):
    deg = dega[:, 0:1] + degb[:, 0:1] + 1.0
    dis = lax.rsqrt(deg)
    x0 = jnp.maximum(jnp.dot(noise[...], w_emb[...],
                             preferred_element_type=jnp.float32) + b_emb[...], 0.0)
    hp = dis * jnp.dot(x0, w1[...], preferred_element_type=jnp.float32)
    hpa[...] = hp[:, :HH]
    hpb[...] = hp[:, HH:]
    dis_o[...] = dis


def _combine_body(sa, sb, dis, b_l, w_next, hpa, hpb):
    svec = jnp.concatenate([sa[...], sb[...]], axis=1)
    x = jnp.maximum(dis[...] * svec + b_l[...], 0.0)
    hp = dis[...] * jnp.dot(x, w_next[...], preferred_element_type=jnp.float32)
    hpa[...] = hp[:, :HH]
    hpb[...] = hp[:, HH:]


def _final_body(sa, sb, dis, b_l, w_out, b_out, ang):
    svec = jnp.concatenate([sa[...], sb[...]], axis=1)
    x = jnp.maximum(dis[...] * svec + b_l[...], 0.0)
    y = jnp.dot(x, w_out[...], preferred_element_type=jnp.float32) + b_out[...]
    ang[...] = jnp.tanh(y) * jnp.pi


def _row_spec(width):
    return pl.BlockSpec((BN, width), lambda i: (i, 0))


def _full_spec(shape):
    return pl.BlockSpec(shape, lambda i: tuple(0 for _ in shape))


def kernel(noise_params, edge_index, W_emb, b_emb, W1, b1, W2, b2, W3, b3, W_out, b_out):
    src = edge_index[0]
    dst = edge_index[1]
    pad = E_PAD - E
    src2 = jnp.concatenate([src, jnp.zeros((pad,), jnp.int32)]).reshape(NB, EB)
    dst3 = jnp.concatenate([dst, jnp.full((pad,), N, jnp.int32)]).reshape(NB, EB)

    dega, degb = _deg_kernel(dst3.reshape(32, BPT_DEG, EB))

    noise = noise_params.reshape(N, 4)
    embed = pl.pallas_call(
        _embed_body,
        grid=(GRID,),
        in_specs=[
            _row_spec(4), _row_spec(16), _row_spec(16),
            _full_spec((4, H)), _full_spec((1, H)), _full_spec((H, H)),
        ],
        out_specs=[_row_spec(HH), _row_spec(HH), _row_spec(1)],
        out_shape=[
            jax.ShapeDtypeStruct((N, HH), jnp.float32),
            jax.ShapeDtypeStruct((N, HH), jnp.float32),
            jax.ShapeDtypeStruct((N, 1), jnp.float32),
        ],
    )
    hpa, hpb, dis = embed(noise, dega, degb, W_emb, b_emb.reshape(1, H), W1)

    combine = pl.pallas_call(
        _combine_body,
        grid=(GRID,),
        in_specs=[
            _row_spec(HH), _row_spec(HH), _row_spec(1),
            _full_spec((1, H)), _full_spec((H, H)),
        ],
        out_specs=[_row_spec(HH), _row_spec(HH)],
        out_shape=[
            jax.ShapeDtypeStruct((N, HH), jnp.float32),
            jax.ShapeDtypeStruct((N, HH), jnp.float32),
        ],
    )

    sa, sb = _msg_kernel(hpa, hpb, src2, dst3)
    hpa, hpb = combine(sa, sb, dis, b1.reshape(1, H), W2)
    sa, sb = _msg_kernel(hpa, hpb, src2, dst3)
    hpa, hpb = combine(sa, sb, dis, b2.reshape(1, H), W3)
    sa, sb = _msg_kernel(hpa, hpb, src2, dst3)

    final = pl.pallas_call(
        _final_body,
        grid=(GRID,),
        in_specs=[
            _row_spec(HH), _row_spec(HH), _row_spec(1),
            _full_spec((1, H)), _full_spec((H, 3)), _full_spec((1, 3)),
        ],
        out_specs=[_row_spec(3)],
        out_shape=[jax.ShapeDtypeStruct((N, 3), jnp.float32)],
    )
    (ang,) = final(sa, sb, dis, b3.reshape(1, H), W_out, b_out.reshape(1, 3))
    return ang.reshape(1, N, 3)


# speedup vs baseline: 26.4273x; 1.0219x over previous
import functools
import jax
import jax.numpy as jnp
from jax import lax
from jax.experimental import pallas as pl
from jax.experimental.pallas import tpu as pltpu
from jax.experimental.pallas import tpu_sc as plsc

N = 50000
H = 64
HH = 32
E = 800000
EB = 128
NB = 6272
E_PAD = NB * EB
NACC = 50008
BPT_MSG = NB // 16
BPT_DEG = NB // 32
GRP = 14
NBUF = 6
ROWS_A = 3128
ROWS_B = N - 15 * ROWS_A

_mesh = plsc.VectorSubcoreMesh(core_axis_name="c", subcore_axis_name="s")
_sc_params = pltpu.CompilerParams(use_tc_tiling_on_sc=False)


@functools.partial(
    pl.kernel,
    out_type=(
        jax.ShapeDtypeStruct((N, 16), jnp.float32),
        jax.ShapeDtypeStruct((N, 16), jnp.float32),
    ),
    mesh=_mesh,
    compiler_params=_sc_params,
    scratch_types=[
        pltpu.VMEM((BPT_DEG, EB), jnp.int32),
        pltpu.VMEM((EB, 16), jnp.float32),
        pltpu.VMEM((ROWS_A // 4, 16), jnp.float32),
        pltpu.VMEM_SHARED((NACC, 16), jnp.float32),
    ],
)
def _deg_kernel(dst4, dega, degb, dstbuf, ones_v, zbuf, acc):
    c = lax.axis_index("c")
    s = lax.axis_index("s")
    wid = c * 16 + s

    def fill(i, _):
        ones_v[i, :] = jnp.full((16,), 1.0, jnp.float32)
        return 0

    lax.fori_loop(0, EB, fill, 0)

    def zfill(i, _):
        zbuf[i, :] = jnp.zeros((16,), jnp.float32)
        return 0

    lax.fori_loop(0, ROWS_A // 4, zfill, 0)
    for k in range(4):
        pltpu.sync_copy(zbuf, acc.at[pl.ds(s * ROWS_A + k * (ROWS_A // 4), ROWS_A // 4)])
    plsc.subcore_barrier()

    pltpu.sync_copy(dst4.at[wid], dstbuf)

    def body(b, _):
        pltpu.sync_copy(ones_v, acc.at[dstbuf.at[b]], add=True)
        return 0

    lax.fori_loop(0, BPT_DEG, body, 0)
    plsc.subcore_barrier()

    def drain(out_ref):
        @pl.when(s < 15)
        def _():
            off = s * ROWS_A
            pltpu.sync_copy(acc.at[pl.ds(off, ROWS_A)], out_ref.at[pl.ds(off, ROWS_A)])

        @pl.when(s == 15)
        def _():
            off = 15 * ROWS_A
            pltpu.sync_copy(acc.at[pl.ds(off, ROWS_B)], out_ref.at[pl.ds(off, ROWS_B)])

    @pl.when(c == 0)
    def _():
        drain(dega)

    @pl.when(c == 1)
    def _():
        drain(degb)


@functools.partial(
    pl.kernel,
    out_type=(
        jax.ShapeDtypeStruct((N, HH), jnp.float32),
        jax.ShapeDtypeStruct((N, HH), jnp.float32),
    ),
    mesh=_mesh,
    compiler_params=_sc_params,
    scratch_types=[
        pltpu.VMEM((GRP, EB), jnp.int32),
        pltpu.VMEM((GRP, EB), jnp.int32),
        [pltpu.VMEM((EB, HH), jnp.float32) for _ in range(NBUF)],
        pltpu.VMEM_SHARED((NACC, HH), jnp.float32),
        pltpu.SemaphoreType.DMA,
        pltpu.SemaphoreType.DMA,
    ],
)
def _msg_kernel(hpa, hpb, src2, dst3, outa, outb, srcbuf, dstbuf, rows, acc, gsem, ssem):
    c = lax.axis_index("c")
    s = lax.axis_index("s")

    def run(hp_ref, out_ref):
        @pl.when(s < 15)
        def _():
            off = s * ROWS_A
            pltpu.sync_copy(hp_ref.at[pl.ds(off, ROWS_A)], acc.at[pl.ds(off, ROWS_A)])

        @pl.when(s == 15)
        def _():
            off = 15 * ROWS_A
            pltpu.sync_copy(hp_ref.at[pl.ds(off, ROWS_B)], acc.at[pl.ds(off, ROWS_B)])

        plsc.subcore_barrier()

        base = s * BPT_MSG

        D = NBUF // 2

        def group(g, _):
            pltpu.sync_copy(src2.at[pl.ds(base + g * GRP, GRP)], srcbuf)
            pltpu.sync_copy(dst3.at[pl.ds(base + g * GRP, GRP)], dstbuf)
            for j in range(D):
                pltpu.async_copy(hp_ref.at[srcbuf.at[j]], rows[j], gsem)
            for b in range(GRP):
                j = b % NBUF
                pltpu.make_async_copy(hp_ref.at[srcbuf.at[b]], rows[j], gsem).wait()
                pltpu.async_copy(rows[j], acc.at[dstbuf.at[b]], ssem, add=True)
                if b >= D:
                    pltpu.make_async_copy(
                        rows[(j + D) % NBUF], acc.at[dstbuf.at[b - D]], ssem).wait()
                if b + D < GRP:
                    pltpu.async_copy(
                        hp_ref.at[srcbuf.at[b + D]], rows[(j + D) % NBUF], gsem)
            for b in range(GRP - D, GRP):
                j = b % NBUF
                pltpu.make_async_copy(rows[j], acc.at[dstbuf.at[b]], ssem).wait()
            return 0

        lax.fori_loop(0, BPT_MSG // GRP, group, 0)
        plsc.subcore_barrier()

        @pl.when(s < 15)
        def _():
            off = s * ROWS_A
            pltpu.sync_copy(acc.at[pl.ds(off, ROWS_A)], out_ref.at[pl.ds(off, ROWS_A)])

        @pl.when(s == 15)
        def _():
            off = 15 * ROWS_A
            pltpu.sync_copy(acc.at[pl.ds(off, ROWS_B)], out_ref.at[pl.ds(off, ROWS_B)])

    @pl.when(c == 0)
    def _():
        run(hpa, outa)

    @pl.when(c == 1)
    def _():
        run(hpb, outb)


BN = 5000
GRID = N // BN


def _embed_body(noise, dega, degb, w_emb, b_emb, w1, hpa, hpb, dis_o):
    deg = dega[:, 0:1] + degb[:, 0:1] + 1.0
    dis = lax.rsqrt(deg)
    x0 = jnp.maximum(jnp.dot(noise[...], w_emb[...],
                             preferred_element_type=jnp.float32) + b_emb[...], 0.0)
    hp = dis * jnp.dot(x0, w1[...], preferred_element_type=jnp.float32)
    hpa[...] = hp[:, :HH]
    hpb[...] = hp[:, HH:]
    dis_o[...] = dis


def _combine_body(sa, sb, dis, b_l, w_next, hpa, hpb):
    svec = jnp.concatenate([sa[...], sb[...]], axis=1)
    x = jnp.maximum(dis[...] * svec + b_l[...], 0.0)
    hp = dis[...] * jnp.dot(x, w_next[...], preferred_element_type=jnp.float32)
    hpa[...] = hp[:, :HH]
    hpb[...] = hp[:, HH:]


def _final_body(sa, sb, dis, b_l, w_out, b_out, ang):
    svec = jnp.concatenate([sa[...], sb[...]], axis=1)
    x = jnp.maximum(dis[...] * svec + b_l[...], 0.0)
    y = jnp.dot(x, w_out[...], preferred_element_type=jnp.float32) + b_out[...]
    ang[...] = jnp.tanh(y) * jnp.pi


def _row_spec(width):
    return pl.BlockSpec((BN, width), lambda i: (i, 0))


def _full_spec(shape):
    return pl.BlockSpec(shape, lambda i: tuple(0 for _ in shape))


def kernel(noise_params, edge_index, W_emb, b_emb, W1, b1, W2, b2, W3, b3, W_out, b_out):
    src = edge_index[0]
    dst = edge_index[1]
    pad = E_PAD - E
    src2 = jnp.concatenate([src, jnp.zeros((pad,), jnp.int32)]).reshape(NB, EB)
    dst3 = jnp.concatenate([dst, jnp.full((pad,), N, jnp.int32)]).reshape(NB, EB)

    dega, degb = _deg_kernel(dst3.reshape(32, BPT_DEG, EB))

    noise = noise_params.reshape(N, 4)
    embed = pl.pallas_call(
        _embed_body,
        grid=(GRID,),
        in_specs=[
            _row_spec(4), _row_spec(16), _row_spec(16),
            _full_spec((4, H)), _full_spec((1, H)), _full_spec((H, H)),
        ],
        out_specs=[_row_spec(HH), _row_spec(HH), _row_spec(1)],
        out_shape=[
            jax.ShapeDtypeStruct((N, HH), jnp.float32),
            jax.ShapeDtypeStruct((N, HH), jnp.float32),
            jax.ShapeDtypeStruct((N, 1), jnp.float32),
        ],
    )
    hpa, hpb, dis = embed(noise, dega, degb, W_emb, b_emb.reshape(1, H), W1)

    combine = pl.pallas_call(
        _combine_body,
        grid=(GRID,),
        in_specs=[
            _row_spec(HH), _row_spec(HH), _row_spec(1),
            _full_spec((1, H)), _full_spec((H, H)),
        ],
        out_specs=[_row_spec(HH), _row_spec(HH)],
        out_shape=[
            jax.ShapeDtypeStruct((N, HH), jnp.float32),
            jax.ShapeDtypeStruct((N, HH), jnp.float32),
        ],
    )

    sa, sb = _msg_kernel(hpa, hpb, src2, dst3)
    hpa, hpb = combine(sa, sb, dis, b1.reshape(1, H), W2)
    sa, sb = _msg_kernel(hpa, hpb, src2, dst3)
    hpa, hpb = combine(sa, sb, dis, b2.reshape(1, H), W3)
    sa, sb = _msg_kernel(hpa, hpb, src2, dst3)

    final = pl.pallas_call(
        _final_body,
        grid=(GRID,),
        in_specs=[
            _row_spec(HH), _row_spec(HH), _row_spec(1),
            _full_spec((1, H)), _full_spec((H, 3)), _full_spec((1, 3)),
        ],
        out_specs=[_row_spec(3)],
        out_shape=[jax.ShapeDtypeStruct((N, 3), jnp.float32)],
    )
    (ang,) = final(sa, sb, dis, b3.reshape(1, H), W_out, b_out.reshape(1, 3))
    return ang.reshape(1, N, 3)
